# Initial kernel scaffold; baseline (speedup 1.0000x reference)
import jax
import jax.numpy as jnp
from jax.experimental import pallas as pl


def kernel(vertices):
    raise NotImplementedError("write your pallas kernel here")


# speedup vs baseline: 1.7342x; 1.7342x over previous
import functools

import jax
import jax.numpy as jnp
import numpy as np
from jax import lax
from jax.experimental import pallas as pl
from jax.experimental.pallas import tpu as pltpu
from jax.experimental.pallas import tpu_sc as plsc

N_RADIAL = 5
N_ANGULAR = 8
RADIUS = 0.1
TEMPLATE_SCALE = 0.75
K = 16
V = 4096
B = 2
NT = N_RADIAL * N_ANGULAR * 3
BLK = 256
NUM_BLKS = V // BLK
NW = 32
VPW = V // NW


def _template_points():
    radius = TEMPLATE_SCALE * RADIUS
    radii = radius * (np.arange(1, N_RADIAL + 1, dtype=np.float32) / N_RADIAL)
    angles = np.linspace(0.0, 2.0 * np.pi, N_ANGULAR,
                         endpoint=False).astype(np.float32)
    r = radii[:, None]
    a = angles[None, :]
    t = np.stack([r * np.cos(a), r * np.sin(a)], axis=-1).astype(np.float32)
    return t.reshape(N_RADIAL * N_ANGULAR, 2)


def _knn_kernel(d2_ref, out_ref):
    d2 = d2_ref[0]
    iota0 = lax.broadcasted_iota(jnp.int32, (V, BLK), 0)
    inf = jnp.float32(jnp.inf)
    for k in range(K):
        m = jnp.min(d2, axis=0, keepdims=True)
        imin = jnp.min(jnp.where(d2 == m, iota0, V), axis=0, keepdims=True)
        out_ref[0, pl.ds(k, 1), :] = imin
        d2 = jnp.where(iota0 == imin, inf, d2)


def _run_knn(d2t):
    return pl.pallas_call(
        _knn_kernel,
        grid=(B, NUM_BLKS),
        in_specs=[pl.BlockSpec((1, V, BLK), lambda b, i: (b, 0, i))],
        out_specs=pl.BlockSpec((1, K, BLK), lambda b, i: (b, 0, i)),
        out_shape=jax.ShapeDtypeStruct((B, K, V), jnp.int32),
    )(d2t)


def _projections(nb):
    d = jnp.sqrt(jnp.sum(nb ** 2, axis=-1) + 1e-16)
    w = jax.nn.relu(RADIUS - d)
    cov = jnp.einsum('vn,vni,vnj->vij', w, nb, nb)
    cov = cov / (jnp.sum(w, axis=-1)[:, None, None] + 1e-12)
    cov = cov + 1e-8 * jnp.eye(3, dtype=cov.dtype)
    _, evecs = jnp.linalg.eigh(cov)
    normal = evecs[..., 0]
    x_ax = evecs[..., 2]
    y_ax = evecs[..., 1]
    lrfs = jnp.stack([normal, x_ax, y_ax], axis=1)
    normals = lrfs[:, 0, :]
    scaled = (nb @ normals[:, :, None]) * normals[:, None, :]
    proj = nb - scaled
    proj = jnp.einsum('vij,vnj->vni',
                      jnp.linalg.inv(jnp.transpose(lrfs, (0, 2, 1))),
                      proj)[:, :, 1:]
    nrm = jnp.sqrt(jnp.sum(nb ** 2, axis=-1) + 1e-16)[..., None]
    proj = proj / jnp.sqrt(jnp.maximum(
        jnp.sum(proj * proj, axis=-1, keepdims=True), 1e-12))
    return nrm * proj


def _match_kernel(px_ref, py_ref, out_ref):
    ppx = px_ref[0]
    ppy = py_ref[0]
    iota = lax.broadcasted_iota(jnp.int32, (BLK, K), 1)
    inf = jnp.float32(jnp.inf)
    tpl = _template_points()
    for p in range(tpl.shape[0]):
        tx = float(tpl[p, 0])
        ty = float(tpl[p, 1])
        dx = tx - ppx
        dy = ty - ppy
        dist = jnp.sqrt(dx * dx + dy * dy + 1e-16)
        for slot in range(3):
            m = jnp.min(dist, axis=1, keepdims=True)
            imin = jnp.min(jnp.where(dist == m, iota, K), axis=1,
                           keepdims=True)
            out_ref[0, :, pl.ds(p * 3 + slot, 1)] = imin
            dist = jnp.where(iota == imin, inf, dist)


def _run_match(px, py):
    return pl.pallas_call(
        _match_kernel,
        grid=(B, NUM_BLKS),
        in_specs=[
            pl.BlockSpec((1, BLK, K), lambda b, i: (b, i, 0)),
            pl.BlockSpec((1, BLK, K), lambda b, i: (b, i, 0)),
        ],
        out_specs=pl.BlockSpec((1, BLK, NT), lambda b, i: (b, i, 0)),
        out_shape=jax.ShapeDtypeStruct((B, V, NT), jnp.int32),
    )(px, py)


def _sc_pid_body(idx_hbm, cls_hbm, out_hbm, ibuf, cbuf, pbuf):
    wid = lax.axis_index("s") * 2 + lax.axis_index("c")
    lane = lax.iota(jnp.int32, 16)
    for b in range(B):
        pltpu.sync_copy(idx_hbm.at[pl.ds((b * V + wid * VPW) * K, VPW * K)],
                        ibuf)
        pltpu.sync_copy(cls_hbm.at[pl.ds((b * V + wid * VPW) * NT, VPW * NT)],
                        cbuf)

        def body(m, carry):
            base = m * NT
            tb = m * K
            for off in (0, 16, 32, 48, 64, 80, 96, NT - 16):
                cvec = cbuf[pl.ds(base + off, 16)]
                got = plsc.load_gather(ibuf, [cvec + tb])
                pbuf[pl.ds(base + off, 16)] = got
            return carry

        lax.fori_loop(0, VPW, body, 0)
        pltpu.sync_copy(pbuf,
                        out_hbm.at[pl.ds((b * V + wid * VPW) * NT, VPW * NT)])


def _run_sc_pid(idx_flat, cls_flat):
    mesh = plsc.VectorSubcoreMesh(core_axis_name="c", subcore_axis_name="s")
    kern = functools.partial(
        pl.kernel,
        mesh=mesh,
        compiler_params=pltpu.CompilerParams(needs_layout_passes=False),
        out_type=jax.ShapeDtypeStruct((B * V * NT,), jnp.int32),
        scratch_types=[
            pltpu.VMEM((VPW * K,), jnp.int32),
            pltpu.VMEM((VPW * NT,), jnp.int32),
            pltpu.VMEM((VPW * NT,), jnp.int32),
        ],
    )(_sc_pid_body)
    return kern(idx_flat, cls_flat)


def kernel(vertices):
    verts = vertices.astype(jnp.float32)

    def _d2(v):
        sq = jnp.sum(v * v, axis=-1)
        return sq[:, None] + sq[None, :] - 2.0 * (v @ v.T)

    d2t = jnp.transpose(jax.vmap(_d2)(verts), (0, 2, 1))
    idx = _run_knn(d2t)
    idx_vmaj = jnp.transpose(idx, (0, 2, 1))

    nb = jax.vmap(lambda v, i: v[i] - v[:, None, :])(verts, idx_vmaj)
    proj = jax.vmap(_projections)(nb)

    out120 = _run_match(proj[..., 0], proj[..., 1])
    closest = jnp.transpose(
        out120.reshape(B, V, N_RADIAL, N_ANGULAR, 3),
        (0, 1, 4, 2, 3))

    pid_flat = _run_sc_pid(idx_vmaj.reshape(B * V * K),
                           out120.reshape(B * V * NT))
    pid = pid_flat.reshape(B, V, N_RADIAL, N_ANGULAR, 3).astype(jnp.float32)

    template = jnp.asarray(
        _template_points().reshape(N_RADIAL, N_ANGULAR, 2))

    def _finish(projections, cls, pidv):
        vi = jnp.arange(V)[:, None, None, None]
        proj_g = projections[vi, cls]
        v0 = proj_g[:, 2] - proj_g[:, 0]
        v1 = proj_g[:, 1] - proj_g[:, 0]
        v2 = template[None] - proj_g[:, 0]
        dot00 = jnp.einsum('vrai,vrai->vra', v0, v0)
        dot01 = jnp.einsum('vrai,vrai->vra', v0, v1)
        dot02 = jnp.einsum('vrai,vrai->vra', v0, v2)
        dot11 = jnp.einsum('vrai,vrai->vra', v1, v1)
        dot12 = jnp.einsum('vrai,vrai->vra', v1, v2)
        denom = dot00 * dot11 - dot01 * dot01 + 1e-12
        p2 = (dot11 * dot02 - dot01 * dot12) / denom
        p1 = (dot00 * dot12 - dot01 * dot02) / denom
        p0 = 1.0 - p2 - p1
        weights = jnp.stack([p2, p1, p0], axis=-1)
        return jnp.stack([pidv, weights], axis=-1)

    return jax.vmap(_finish)(proj, closest, pid)
